# Initial kernel scaffold; baseline (speedup 1.0000x reference)
import jax
import jax.numpy as jnp
from jax.experimental import pallas as pl


def kernel(x, edge_index, edge_type, batch, W, b, fc1_W, fc1_b, fc2_W, fc2_b, bn_gamma, bn_beta):
    raise NotImplementedError("write your pallas kernel here")


# speedup vs baseline: 4.4670x; 4.4670x over previous
import jax
import jax.numpy as jnp
from jax import lax
from jax.experimental import pallas as pl
from jax.experimental.pallas import tpu as pltpu
from jax.experimental.pallas import tpu_sc as plsc

f32 = jnp.float32
i32 = jnp.int32

N_NODES = 10000
D = 128
NG = 64
NCLS = 18
NSC = 2
NT = 16
NW = NSC * NT
N_PAD = 10240
ROWS_PER_TILE = N_PAD // NT
CHUNK = 128
EPT_CHUNKS = 80
EPT = EPT_CHUNKS * CHUNK
E_PAD = NW * EPT
ROW_BLK = 512
N_BLKS = N_PAD // ROW_BLK

_SC_MESH = plsc.VectorSubcoreMesh(core_axis_name="c", subcore_axis_name="s")
_SC_PARAMS = pltpu.CompilerParams(needs_layout_passes=False)


def _spmm_body(g_hbm, src_hbm, dst_hbm, w_hbm, out,
               acc, src_v, dst_v, w_v, rows_v, zero_v, gsem):
    c = lax.axis_index("c")
    s = lax.axis_index("s")
    wid = c * NT + s

    def fzero(i, carry):
        for k in range(D // 16):
            zero_v[i, pl.ds(k * 16, 16)] = jnp.zeros((16,), f32)
        return carry
    lax.fori_loop(0, 16, fzero, 0)

    def zbody(z, carry):
        pltpu.sync_copy(zero_v, acc.at[pl.ds(s * ROWS_PER_TILE + z * 16, 16)])
        return carry
    lax.fori_loop(0, ROWS_PER_TILE // 16, zbody, 0)
    plsc.subcore_barrier()

    def chunk_body(ch, carry):
        pltpu.async_copy(g_hbm.at[src_v.at[ch]], rows_v, gsem).wait()

        def mul_body(jg, carry2):
            wgroup = w_v[ch, pl.ds(jg * 16, 16)]
            for l in range(16):
                j = jg * 16 + l
                wv = jnp.full((16,), wgroup[l], f32)
                for k in range(8):
                    sl = pl.ds(k * 16, 16)
                    rows_v[j, sl] = rows_v[j, sl] * wv
            return carry2
        lax.fori_loop(0, CHUNK // 16, mul_body, 0)

        pltpu.sync_copy(rows_v, acc.at[dst_v.at[ch]], add=True)
        return carry

    for h in range(2):
        hs = pl.ds(h * (EPT_CHUNKS // 2), EPT_CHUNKS // 2)
        pltpu.sync_copy(src_hbm.at[wid, hs], src_v)
        pltpu.sync_copy(dst_hbm.at[wid, hs], dst_v)
        pltpu.sync_copy(w_hbm.at[wid, hs], w_v)
        lax.fori_loop(0, EPT_CHUNKS // 2, chunk_body, 0)
    plsc.subcore_barrier()

    rs = pl.ds(s * ROWS_PER_TILE, ROWS_PER_TILE)
    pltpu.sync_copy(acc.at[rs], out.at[c, rs])


def _spmm(g, src3, dst3, w3):
    return pl.kernel(
        _spmm_body,
        out_type=jax.ShapeDtypeStruct((NSC, N_PAD, D), f32),
        mesh=_SC_MESH,
        compiler_params=_SC_PARAMS,
        scratch_types=[
            pltpu.VMEM_SHARED((N_PAD, D), f32),
            pltpu.VMEM((EPT_CHUNKS // 2, CHUNK), i32),
            pltpu.VMEM((EPT_CHUNKS // 2, CHUNK), i32),
            pltpu.VMEM((EPT_CHUNKS // 2, CHUNK), f32),
            pltpu.VMEM((CHUNK, D), f32),
            pltpu.VMEM((16, D), f32),
            pltpu.SemaphoreType.DMA,
        ],
    )(g, src3, dst3, w3)


def _prep_body(x_ref, w0_ref, d0_ref, d1_ref, dinv_ref, g0_ref):
    deg = 1.0 + d0_ref[0, :, :1] + d1_ref[0, :, :1]
    dinv = lax.rsqrt(deg)
    dinv_ref[...] = dinv
    g0_ref[...] = dinv * jnp.dot(x_ref[...], w0_ref[...],
                                 preferred_element_type=f32)


def _prep(x_pad, w0, degp):
    return pl.pallas_call(
        _prep_body,
        grid=(N_BLKS,),
        in_specs=[
            pl.BlockSpec((ROW_BLK, D), lambda i: (i, 0)),
            pl.BlockSpec((D, D), lambda i: (0, 0)),
            pl.BlockSpec((1, ROW_BLK, D), lambda i: (0, i, 0)),
            pl.BlockSpec((1, ROW_BLK, D), lambda i: (1, i, 0)),
        ],
        out_specs=[
            pl.BlockSpec((ROW_BLK, 1), lambda i: (i, 0)),
            pl.BlockSpec((ROW_BLK, D), lambda i: (i, 0)),
        ],
        out_shape=[jax.ShapeDtypeStruct((N_PAD, 1), f32),
                   jax.ShapeDtypeStruct((N_PAD, D), f32)],
    )(x_pad, w0, degp, degp)


def _layer_body(p0_ref, p1_ref, g_ref, dinv_ref, b_ref, w_ref, out_ref):
    dv = dinv_ref[...]
    h = dv * (p0_ref[0] + p1_ref[0] + g_ref[...]) + b_ref[...]
    h = jnp.maximum(h, 0.0)
    out_ref[...] = dv * jnp.dot(h, w_ref[...], preferred_element_type=f32)


def _layer(p, g, dinv, bvec, w):
    return pl.pallas_call(
        _layer_body,
        grid=(N_BLKS,),
        in_specs=[
            pl.BlockSpec((1, ROW_BLK, D), lambda i: (0, i, 0)),
            pl.BlockSpec((1, ROW_BLK, D), lambda i: (1, i, 0)),
            pl.BlockSpec((ROW_BLK, D), lambda i: (i, 0)),
            pl.BlockSpec((ROW_BLK, 1), lambda i: (i, 0)),
            pl.BlockSpec((1, D), lambda i: (0, 0)),
            pl.BlockSpec((D, D), lambda i: (0, 0)),
        ],
        out_specs=pl.BlockSpec((ROW_BLK, D), lambda i: (i, 0)),
        out_shape=jax.ShapeDtypeStruct((N_PAD, D), f32),
    )(p, p, g, dinv, bvec, w)


def _pool_body(p0_ref, p1_ref, g_ref, dinv_ref, b_ref, batch_ref, out_ref):
    i = pl.program_id(0)
    h = dinv_ref[...] * (p0_ref[0] + p1_ref[0] + g_ref[...]) + b_ref[...]
    onehot = (lax.broadcasted_iota(i32, (NG, ROW_BLK), 0)
              == batch_ref[...]).astype(f32)
    acc = jnp.dot(onehot, h, preferred_element_type=f32)

    @pl.when(i == 0)
    def _():
        out_ref[...] = acc

    @pl.when(i != 0)
    def _():
        out_ref[...] += acc


def _pool(p, g, dinv, bvec, batch2):
    return pl.pallas_call(
        _pool_body,
        grid=(N_BLKS,),
        in_specs=[
            pl.BlockSpec((1, ROW_BLK, D), lambda i: (0, i, 0)),
            pl.BlockSpec((1, ROW_BLK, D), lambda i: (1, i, 0)),
            pl.BlockSpec((ROW_BLK, D), lambda i: (i, 0)),
            pl.BlockSpec((ROW_BLK, 1), lambda i: (i, 0)),
            pl.BlockSpec((1, D), lambda i: (0, 0)),
            pl.BlockSpec((1, ROW_BLK), lambda i: (0, i)),
        ],
        out_specs=pl.BlockSpec((NG, D), lambda i: (0, 0)),
        out_shape=jax.ShapeDtypeStruct((NG, D), f32),
    )(p, p, g, dinv, bvec, batch2)


def _head_body(pooled_ref, fc1w_ref, fc1b_ref, fc2w_ref, fc2b_ref,
               gam_ref, bet_ref, out_ref):
    p = pooled_ref[...]
    mean = jnp.mean(p, axis=0, keepdims=True)
    var = jnp.mean((p - mean) ** 2, axis=0, keepdims=True)
    hn = (p - mean) * lax.rsqrt(var + 1e-5) * gam_ref[...] + bet_ref[...]
    hf = jnp.dot(hn, fc1w_ref[...], preferred_element_type=f32) + fc1b_ref[...]
    hf = jnp.maximum(hf, 0.0)
    logits = jnp.dot(hf, fc2w_ref[...], preferred_element_type=f32) + fc2b_ref[...]
    col = lax.broadcasted_iota(i32, (NG, D), 1)
    lm = jnp.where(col < NCLS, logits, -1e30)
    mx = jnp.max(lm, axis=1, keepdims=True)
    lse = jnp.log(jnp.sum(jnp.exp(lm - mx), axis=1, keepdims=True)) + mx
    out_ref[...] = logits - lse


def _head(pooled, fc1w, fc1b2, fc2w_pad, fc2b_pad, gam2, bet2):
    return pl.pallas_call(
        _head_body,
        out_shape=jax.ShapeDtypeStruct((NG, D), f32),
    )(pooled, fc1w, fc1b2, fc2w_pad, fc2b_pad, gam2, bet2)


def kernel(x, edge_index, edge_type, batch, W, b,
           fc1_W, fc1_b, fc2_W, fc2_b, bn_gamma, bn_beta):
    n = x.shape[0]
    ne = edge_index.shape[1]
    n_conv = W.shape[0]

    src = edge_index[0].astype(i32)
    dst = edge_index[1].astype(i32)
    w = edge_type.astype(f32)

    epad = E_PAD - ne
    src3 = jnp.concatenate([src, jnp.zeros((epad,), i32)]).reshape(NW, EPT_CHUNKS, CHUNK)
    dst3 = jnp.concatenate([dst, jnp.zeros((epad,), i32)]).reshape(NW, EPT_CHUNKS, CHUNK)
    w3 = jnp.concatenate([w, jnp.zeros((epad,), f32)]).reshape(NW, EPT_CHUNKS, CHUNK)

    x_pad = jnp.concatenate([x.astype(f32), jnp.zeros((N_PAD - n, D), f32)])
    batch2 = jnp.concatenate([batch.astype(i32),
                              jnp.full((N_PAD - n,), NG, i32)]).reshape(1, N_PAD)

    ones_g = jnp.ones((N_PAD, D), f32)
    degp = _spmm(ones_g, src3, dst3, w3)
    dinv, g = _prep(x_pad, W[0], degp)

    p = None
    for i in range(n_conv):
        p = _spmm(g, src3, dst3, w3)
        if i + 1 < n_conv:
            g = _layer(p, g, dinv, b[i].reshape(1, D), W[i + 1])

    pooled = _pool(p, g, dinv, b[n_conv - 1].reshape(1, D), batch2)

    fc2w_pad = jnp.zeros((D, D), f32).at[:, :NCLS].set(fc2_W.astype(f32))
    fc2b_pad = jnp.zeros((1, D), f32).at[0, :NCLS].set(fc2_b.astype(f32))
    out = _head(pooled, fc1_W.astype(f32), fc1_b.reshape(1, D).astype(f32),
                fc2w_pad, fc2b_pad, bn_gamma.reshape(1, D).astype(f32),
                bn_beta.reshape(1, D).astype(f32))
    return out[:, :NCLS]


# speedup vs baseline: 6.2917x; 1.4085x over previous
import jax
import jax.numpy as jnp
from jax import lax
from jax.experimental import pallas as pl
from jax.experimental.pallas import tpu as pltpu
from jax.experimental.pallas import tpu_sc as plsc

f32 = jnp.float32
i32 = jnp.int32

N_NODES = 10000
D = 128
NG = 64
NCLS = 18
NSC = 2
NT = 16
NW = NSC * NT
N_PAD = 10240
ROWS_PER_TILE = N_PAD // NT
CHUNK = 128
EPT_CHUNKS = 80
HALF = EPT_CHUNKS // 2
EPT = EPT_CHUNKS * CHUNK
E_PAD = NW * EPT
ROW_BLK = 512
N_BLKS = N_PAD // ROW_BLK

_SC_MESH = plsc.VectorSubcoreMesh(core_axis_name="c", subcore_axis_name="s")
_SC_PARAMS = pltpu.CompilerParams(needs_layout_passes=False)


def _spmm_body(g_hbm, src_hbm, dst_hbm, w_hbm, out,
               acc, src_v, dst_v, w_v, rows0, rows1,
               gsem0, gsem1, ssem0, ssem1):
    c = lax.axis_index("c")
    s = lax.axis_index("s")
    wid = c * NT + s
    rows = (rows0, rows1)
    gsem = (gsem0, gsem1)
    ssem = (ssem0, ssem1)

    def fzero(i, carry):
        for k in range(D // 16):
            rows0[i, pl.ds(k * 16, 16)] = jnp.zeros((16,), f32)
        return carry
    lax.fori_loop(0, CHUNK, fzero, 0)

    def zbody(z, carry):
        pltpu.sync_copy(rows0, acc.at[pl.ds(s * ROWS_PER_TILE + z * CHUNK, CHUNK)])
        return carry
    lax.fori_loop(0, ROWS_PER_TILE // CHUNK, zbody, 0)
    plsc.subcore_barrier()

    def mul(ch, rows_b):
        def mul_body(jg, carry2):
            wgroup = w_v[ch, pl.ds(jg * 16, 16)]
            for l in range(16):
                j = jg * 16 + l
                wv = jnp.full((16,), wgroup[l], f32)
                for k in range(8):
                    sl = pl.ds(k * 16, 16)
                    rows_b[j, sl] = rows_b[j, sl] * wv
            return carry2
        lax.fori_loop(0, CHUNK // 16, mul_body, 0)

    for h in range(2):
        hs = pl.ds(h * HALF, HALF)
        pltpu.sync_copy(src_hbm.at[wid, hs], src_v)
        pltpu.sync_copy(dst_hbm.at[wid, hs], dst_v)
        pltpu.sync_copy(w_hbm.at[wid, hs], w_v)
        for b in range(2):
            pltpu.async_copy(g_hbm.at[src_v.at[b]], rows[b], gsem[b])

        def pair(p, carry):
            for b in range(2):
                ch = p * 2 + b
                pltpu.make_async_copy(g_hbm.at[src_v.at[ch]], rows[b],
                                      gsem[b]).wait()
                mul(ch, rows[b])
                pltpu.async_copy(rows[b], acc.at[dst_v.at[ch]], ssem[b],
                                 add=True)
                nxt = ch + 2

                @pl.when(nxt < HALF)
                def _():
                    pltpu.make_async_copy(rows[b], acc.at[dst_v.at[ch]],
                                          ssem[b]).wait()
                    pltpu.async_copy(g_hbm.at[src_v.at[nxt]], rows[b], gsem[b])
            return carry
        lax.fori_loop(0, HALF // 2, pair, 0)
        for b in range(2):
            pltpu.make_async_copy(rows[b], acc.at[dst_v.at[HALF - 2 + b]],
                                  ssem[b]).wait()
    plsc.subcore_barrier()

    rs = pl.ds(s * ROWS_PER_TILE, ROWS_PER_TILE)
    pltpu.sync_copy(acc.at[rs], out.at[c, rs])


def _spmm(g, src3, dst3, w3):
    return pl.kernel(
        _spmm_body,
        out_type=jax.ShapeDtypeStruct((NSC, N_PAD, D), f32),
        mesh=_SC_MESH,
        compiler_params=_SC_PARAMS,
        scratch_types=[
            pltpu.VMEM_SHARED((N_PAD, D), f32),
            pltpu.VMEM((HALF, CHUNK), i32),
            pltpu.VMEM((HALF, CHUNK), i32),
            pltpu.VMEM((HALF, CHUNK), f32),
            pltpu.VMEM((CHUNK, D), f32),
            pltpu.VMEM((CHUNK, D), f32),
            pltpu.SemaphoreType.DMA,
            pltpu.SemaphoreType.DMA,
            pltpu.SemaphoreType.DMA,
            pltpu.SemaphoreType.DMA,
        ],
    )(g, src3, dst3, w3)


def _deg_body(dst_hbm, w_hbm, out, degp, dst_v, w_v):
    c = lax.axis_index("c")
    s = lax.axis_index("s")
    wid = c * NT + s

    def dz(i, carry):
        degp[pl.ds(i * 16, 16)] = jnp.zeros((16,), f32)
        return carry
    lax.fori_loop(0, N_PAD // 16, dz, 0)

    pltpu.sync_copy(dst_hbm.at[wid], dst_v)
    pltpu.sync_copy(w_hbm.at[wid], w_v)

    def eb(r, carry):
        for gi in range(8):
            sl = pl.ds(gi * 16, 16)
            plsc.addupdate_scatter(degp, [dst_v[r, sl]], w_v[r, sl])
        return carry
    lax.fori_loop(0, EPT_CHUNKS, eb, 0)

    pltpu.sync_copy(degp, out.at[wid])


def _deg(dst3, w3):
    return pl.kernel(
        _deg_body,
        out_type=jax.ShapeDtypeStruct((NW, N_PAD), f32),
        mesh=_SC_MESH,
        compiler_params=_SC_PARAMS,
        scratch_types=[
            pltpu.VMEM((N_PAD,), f32),
            pltpu.VMEM((EPT_CHUNKS, CHUNK), i32),
            pltpu.VMEM((EPT_CHUNKS, CHUNK), f32),
        ],
    )(dst3, w3)


def _dred_body(degp_ref, out_ref):
    out_ref[...] = lax.rsqrt(1.0 + jnp.sum(degp_ref[...], axis=0,
                                           keepdims=True))


def _dred(degp):
    return pl.pallas_call(
        _dred_body,
        grid=(N_BLKS,),
        in_specs=[pl.BlockSpec((NW, ROW_BLK), lambda i: (0, i))],
        out_specs=pl.BlockSpec((1, ROW_BLK), lambda i: (0, i)),
        out_shape=jax.ShapeDtypeStruct((1, N_PAD), f32),
    )(degp)


def _g0_body(x_ref, w0_ref, dinv_ref, g0_ref):
    g0_ref[...] = dinv_ref[...] * jnp.dot(x_ref[...], w0_ref[...],
                                          preferred_element_type=f32)


def _g0(x_pad, w0, dinv):
    return pl.pallas_call(
        _g0_body,
        grid=(N_BLKS,),
        in_specs=[
            pl.BlockSpec((ROW_BLK, D), lambda i: (i, 0)),
            pl.BlockSpec((D, D), lambda i: (0, 0)),
            pl.BlockSpec((ROW_BLK, 1), lambda i: (i, 0)),
        ],
        out_specs=pl.BlockSpec((ROW_BLK, D), lambda i: (i, 0)),
        out_shape=jax.ShapeDtypeStruct((N_PAD, D), f32),
    )(x_pad, w0, dinv)


def _layer_body(p0_ref, p1_ref, g_ref, dinv_ref, b_ref, w_ref, out_ref):
    dv = dinv_ref[...]
    h = dv * (p0_ref[0] + p1_ref[0] + g_ref[...]) + b_ref[...]
    h = jnp.maximum(h, 0.0)
    out_ref[...] = dv * jnp.dot(h, w_ref[...], preferred_element_type=f32)


def _layer(p, g, dinv, bvec, w):
    return pl.pallas_call(
        _layer_body,
        grid=(N_BLKS,),
        in_specs=[
            pl.BlockSpec((1, ROW_BLK, D), lambda i: (0, i, 0)),
            pl.BlockSpec((1, ROW_BLK, D), lambda i: (1, i, 0)),
            pl.BlockSpec((ROW_BLK, D), lambda i: (i, 0)),
            pl.BlockSpec((ROW_BLK, 1), lambda i: (i, 0)),
            pl.BlockSpec((1, D), lambda i: (0, 0)),
            pl.BlockSpec((D, D), lambda i: (0, 0)),
        ],
        out_specs=pl.BlockSpec((ROW_BLK, D), lambda i: (i, 0)),
        out_shape=jax.ShapeDtypeStruct((N_PAD, D), f32),
    )(p, p, g, dinv, bvec, w)


def _pool_body(p0_ref, p1_ref, g_ref, dinv_ref, b_ref, batch_ref, out_ref):
    i = pl.program_id(0)
    h = dinv_ref[...] * (p0_ref[0] + p1_ref[0] + g_ref[...]) + b_ref[...]
    onehot = (lax.broadcasted_iota(i32, (NG, ROW_BLK), 0)
              == batch_ref[...]).astype(f32)
    acc = jnp.dot(onehot, h, preferred_element_type=f32)

    @pl.when(i == 0)
    def _():
        out_ref[...] = acc

    @pl.when(i != 0)
    def _():
        out_ref[...] += acc


def _pool(p, g, dinv, bvec, batch2):
    return pl.pallas_call(
        _pool_body,
        grid=(N_BLKS,),
        in_specs=[
            pl.BlockSpec((1, ROW_BLK, D), lambda i: (0, i, 0)),
            pl.BlockSpec((1, ROW_BLK, D), lambda i: (1, i, 0)),
            pl.BlockSpec((ROW_BLK, D), lambda i: (i, 0)),
            pl.BlockSpec((ROW_BLK, 1), lambda i: (i, 0)),
            pl.BlockSpec((1, D), lambda i: (0, 0)),
            pl.BlockSpec((1, ROW_BLK), lambda i: (0, i)),
        ],
        out_specs=pl.BlockSpec((NG, D), lambda i: (0, 0)),
        out_shape=jax.ShapeDtypeStruct((NG, D), f32),
    )(p, p, g, dinv, bvec, batch2)


def _head_body(pooled_ref, fc1w_ref, fc1b_ref, fc2w_ref, fc2b_ref,
               gam_ref, bet_ref, out_ref):
    p = pooled_ref[...]
    mean = jnp.mean(p, axis=0, keepdims=True)
    var = jnp.mean((p - mean) ** 2, axis=0, keepdims=True)
    hn = (p - mean) * lax.rsqrt(var + 1e-5) * gam_ref[...] + bet_ref[...]
    hf = jnp.dot(hn, fc1w_ref[...], preferred_element_type=f32) + fc1b_ref[...]
    hf = jnp.maximum(hf, 0.0)
    logits = jnp.dot(hf, fc2w_ref[...], preferred_element_type=f32) + fc2b_ref[...]
    col = lax.broadcasted_iota(i32, (NG, D), 1)
    lm = jnp.where(col < NCLS, logits, -1e30)
    mx = jnp.max(lm, axis=1, keepdims=True)
    lse = jnp.log(jnp.sum(jnp.exp(lm - mx), axis=1, keepdims=True)) + mx
    out_ref[...] = logits - lse


def _head(pooled, fc1w, fc1b2, fc2w_pad, fc2b_pad, gam2, bet2):
    return pl.pallas_call(
        _head_body,
        out_shape=jax.ShapeDtypeStruct((NG, D), f32),
    )(pooled, fc1w, fc1b2, fc2w_pad, fc2b_pad, gam2, bet2)


def kernel(x, edge_index, edge_type, batch, W, b,
           fc1_W, fc1_b, fc2_W, fc2_b, bn_gamma, bn_beta):
    n = x.shape[0]
    ne = edge_index.shape[1]
    n_conv = W.shape[0]

    src = edge_index[0].astype(i32)
    dst = edge_index[1].astype(i32)
    w = edge_type.astype(f32)

    epad = E_PAD - ne
    src3 = jnp.concatenate([src, jnp.zeros((epad,), i32)]).reshape(NW, EPT_CHUNKS, CHUNK)
    dst3 = jnp.concatenate([dst, jnp.zeros((epad,), i32)]).reshape(NW, EPT_CHUNKS, CHUNK)
    w3 = jnp.concatenate([w, jnp.zeros((epad,), f32)]).reshape(NW, EPT_CHUNKS, CHUNK)

    x_pad = jnp.concatenate([x.astype(f32), jnp.zeros((N_PAD - n, D), f32)])
    batch2 = jnp.concatenate([batch.astype(i32),
                              jnp.full((N_PAD - n,), NG, i32)]).reshape(1, N_PAD)

    degp = _deg(dst3, w3)
    dinv = _dred(degp).reshape(N_PAD, 1)
    g = _g0(x_pad, W[0], dinv)

    p = None
    for i in range(n_conv):
        p = _spmm(g, src3, dst3, w3)
        if i + 1 < n_conv:
            g = _layer(p, g, dinv, b[i].reshape(1, D), W[i + 1])

    pooled = _pool(p, g, dinv, b[n_conv - 1].reshape(1, D), batch2)

    fc2w_pad = jnp.zeros((D, D), f32).at[:, :NCLS].set(fc2_W.astype(f32))
    fc2b_pad = jnp.zeros((1, D), f32).at[0, :NCLS].set(fc2_b.astype(f32))
    out = _head(pooled, fc1_W.astype(f32), fc1_b.reshape(1, D).astype(f32),
                fc2w_pad, fc2b_pad, bn_gamma.reshape(1, D).astype(f32),
                bn_beta.reshape(1, D).astype(f32))
    return out[:, :NCLS]
